# Initial kernel scaffold; baseline (speedup 1.0000x reference)
import jax
import jax.numpy as jnp
from jax.experimental import pallas as pl


def kernel(x, table0, table1):
    raise NotImplementedError("write your pallas kernel here")


# speedup vs baseline: 844.3172x; 844.3172x over previous
import functools
import jax
import jax.numpy as jnp
from jax import lax
from jax.experimental import pallas as pl
from jax.experimental.pallas import tpu as pltpu, tpu_sc as plsc

_NUM_BITS = 16
_C = 2
_SCALE = 2e-05
_BITS_PER_CHUNK = _NUM_BITS // _C
_CHUNK_SIZE = 2 ** _BITS_PER_CHUNK
_TOTAL_SIZE = 2 ** _NUM_BITS
_MASK = _CHUNK_SIZE - 1

_NC = 2
_NS = 16
_NW = _NC * _NS
_L = 16

_MAGIC = float(2.0 ** 23)


def _make_sc_kernel(n_total: int, chunk: int):
  per_w = n_total // _NW
  n_chunks = per_w // chunk
  n_vecs = chunk // _L
  mesh = plsc.VectorSubcoreMesh(core_axis_name="c", subcore_axis_name="s",
                                num_cores=_NC, num_subcores=_NS)

  @functools.partial(
      pl.kernel,
      out_type=jax.ShapeDtypeStruct((n_total,), jnp.float32),
      mesh=mesh,
      compiler_params=pltpu.CompilerParams(needs_layout_passes=False),
      scratch_types=[
          pltpu.VMEM((_CHUNK_SIZE,), jnp.float32),
          pltpu.VMEM((_CHUNK_SIZE,), jnp.float32),
          pltpu.VMEM((chunk,), jnp.float32),
          pltpu.VMEM((chunk,), jnp.float32),
      ],
  )
  def k(x_hbm, t0_hbm, t1_hbm, out_hbm, t0_v, t1_v, xbuf, obuf):
    wid = lax.axis_index("s") * _NC + lax.axis_index("c")
    base = wid * per_w
    pltpu.sync_copy(t0_hbm, t0_v)
    pltpu.sync_copy(t1_hbm, t1_v)

    def chunk_body(c, _):
      off = base + c * chunk
      pltpu.sync_copy(x_hbm.at[pl.ds(off, chunk)], xbuf)

      def vec_body(i, _):
        xv = xbuf[pl.ds(i * _L, _L)]
        y = xv / jnp.float32(_SCALE)
        y = jnp.minimum(jnp.maximum(y, jnp.float32(0.0)),
                        jnp.float32(_TOTAL_SIZE - 1))
        y = (y + jnp.float32(_MAGIC)) - jnp.float32(_MAGIC)
        idx = y.astype(jnp.int32)
        lo = jnp.bitwise_and(idx, _MASK)
        hi = lax.shift_right_logical(idx, _BITS_PER_CHUNK)
        v = plsc.load_gather(t0_v, [lo]) + plsc.load_gather(t1_v, [hi])
        obuf[pl.ds(i * _L, _L)] = v
        return 0

      lax.fori_loop(0, n_vecs, vec_body, 0)
      pltpu.sync_copy(obuf, out_hbm.at[pl.ds(off, chunk)])
      return 0

    lax.fori_loop(0, n_chunks, chunk_body, 0)

  return k


@jax.jit
def kernel(x, table0, table1):
  shape = x.shape
  n_total = x.size
  flat = x.reshape((n_total,))
  out = _make_sc_kernel(n_total, chunk=16384)(flat, table0, table1)
  return out.reshape(shape)
